# baseline (speedup 1.0000x reference)
import functools

import jax
import jax.numpy as jnp
from jax import lax
from jax.experimental import pallas as pl
from jax.experimental.pallas import tpu as pltpu
from jax.experimental.pallas import tpu_sc as plsc

L = 16
NCH = 8
DIM = 128
RPT = 24
PART = (NCH + 1) * L
P3_TILE = 14
_FETCH = 10 * PART
_SP1 = 13 * PART + _FETCH
_PIE_OFF = 16 * L
_SP2 = _PIE_OFF + 3 * DIM


def _sc_body(idx_hbm, w_hbm, head_hbm, out_hbm,
             idx_v, rows_v, part_v, pie_v, slab_v, part2_v,
             slab2_v, head_v, out_v, spmem1, spmem2, sem):
  tid = lax.axis_index("s")
  cid = lax.axis_index("c")
  zero = jnp.zeros((L,), jnp.float32)
  isp3 = jnp.logical_and(tid == P3_TILE, cid == 0)

  cnt = jnp.where(tid < 10, 20,
                  jnp.where(tid < 12, 17,
                            jnp.where(tid == 12, 16,
                                      jnp.where(tid == 13, 20, 0))))
  lo = jnp.where(tid < 10, 0, jnp.where(tid < 13, 10, 13))
  grp = jnp.where(tid < 10, 10,
                  jnp.where(tid < 13, 3, jnp.where(tid == 13, 1, 0)))
  is_leader = jnp.logical_or(tid == 0, jnp.logical_or(tid == 10, tid == 13))
  leader_ix = jnp.where(tid == 0, 0, jnp.where(tid == 10, 1, 2))

  @pl.when(isp3)
  def _():
    pltpu.sync_copy(head_hbm, head_v)

  pltpu.sync_copy(idx_hbm.at[pl.ds(tid * RPT, RPT)], idx_v)
  pltpu.async_copy(w_hbm.at[idx_v], rows_v, sem).wait()

  def p1_body(r, carry):
    ch = [rows_v[r, pl.ds(c * L, L)] for c in range(NCH)]
    s = tuple(carry[c] + ch[c] for c in range(NCH))
    q = carry[NCH]
    for c in range(NCH):
      q = q + ch[c] * ch[c]
    return s + (q,)

  @pl.when(cnt > 0)
  def _():
    carry = lax.fori_loop(0, cnt, p1_body, (zero,) * (NCH + 1))
    for c in range(NCH + 1):
      part_v[pl.ds(c * L, L)] = carry[c]
    pltpu.sync_copy(part_v, spmem1.at[pl.ds(tid * PART, PART)])

  plsc.subcore_barrier()

  def rsqrt16(x):
    i = plsc.bitcast(x, jnp.int32)
    i = jnp.int32(0x5F3759DF) - lax.shift_right_logical(i, 1)
    y = plsc.bitcast(i, jnp.float32)
    for _ in range(3):
      y = y * (1.5 - 0.5 * x * y * y)
    return y

  def exp_dot(rv, r, p):
    d = rv[r, pl.ds(0, L)] * p[0]
    for c in range(1, NCH):
      d = d + rv[r, pl.ds(c * L, L)] * p[c]
    return jnp.exp(jnp.broadcast_to(jnp.sum(d), (L,)))

  @pl.when(cnt > 0)
  def _():
    pltpu.sync_copy(spmem1.at[pl.ds(lo * PART, _FETCH)], slab_v)

    def pie_body(s_, carry):
      return tuple(carry[c] + slab_v[pl.ds(s_ * PART + c * L, L)]
                   for c in range(NCH + 1))

    tot = lax.fori_loop(0, grp, pie_body, (zero,) * (NCH + 1))
    rinv = rsqrt16(jnp.broadcast_to(jnp.sum(tot[NCH]), (L,)))
    pie = tuple(tot[c] * rinv for c in range(NCH))

    def es_body(i, a):
      return (a + exp_dot(rows_v, 2 * i, pie)
              + exp_dot(rows_v, 2 * i + 1, pie))

    acc = lax.fori_loop(0, cnt // 2, es_body, zero)
    tail = exp_dot(rows_v, cnt - 1, pie)
    acc = acc + jnp.where(jnp.broadcast_to(cnt % 2 == 1, (L,)), tail, zero)
    part2_v[...] = acc
    cp = pltpu.async_copy(part2_v, spmem2.at[pl.ds(tid * L, L)], sem)

    @pl.when(is_leader)
    def _():
      for c in range(NCH):
        pie_v[pl.ds(c * L, L)] = pie[c]
      pltpu.sync_copy(pie_v, spmem2.at[pl.ds(_PIE_OFF + leader_ix * DIM, DIM)])

    cp.wait()

  plsc.subcore_barrier()

  @pl.when(isp3)
  def _():
    pltpu.sync_copy(spmem2, slab2_v)
    dn = slab2_v[pl.ds(0, L)]
    for t in range(1, 10):
      dn = dn + slab2_v[pl.ds(t * L, L)]
    dp = (slab2_v[pl.ds(10 * L, L)] + slab2_v[pl.ds(11 * L, L)]
          + slab2_v[pl.ds(12 * L, L)])
    de = slab2_v[pl.ds(13 * L, L)]

    def numer(li):
      pl_ = tuple(slab2_v[pl.ds(_PIE_OFF + li * DIM + c * L, L)]
                  for c in range(NCH))
      return exp_dot(rows_v, 0, pl_)

    pn = numer(0) / dn
    pp = numer(1) / dp
    pe = numer(2) / de
    lane = lax.iota(jnp.int32, L)
    one = zero + 1.0
    pvec = jnp.where(lane == 0, pn,
                     jnp.where(lane == 1, pp,
                               jnp.where(lane == 2, pe,
                                         jnp.where(lane == 3, one, zero))))
    gw = jnp.broadcast_to(jnp.sum(pvec * head_v[...]), (L,))
    out_v[pl.ds(0, L)] = gw
    out_v[pl.ds(L, L)] = 1.0 - gw
    pltpu.sync_copy(out_v, out_hbm)


_sc_kernel = functools.partial(
    pl.kernel,
    out_type=(jax.ShapeDtypeStruct((2 * L,), jnp.float32),),
    mesh=plsc.VectorSubcoreMesh(core_axis_name="c", subcore_axis_name="s",
                                num_cores=1, num_subcores=16),
    scratch_types=[
        pltpu.VMEM((RPT,), jnp.int32),
        pltpu.VMEM((RPT, DIM), jnp.float32),
        pltpu.VMEM((PART,), jnp.float32),
        pltpu.VMEM((DIM,), jnp.float32),
        pltpu.VMEM((_FETCH,), jnp.float32),
        pltpu.VMEM((L,), jnp.float32),
        pltpu.VMEM((_SP2,), jnp.float32),
        pltpu.VMEM((L,), jnp.float32),
        pltpu.VMEM((2 * L,), jnp.float32),
        pltpu.VMEM_SHARED((_SP1,), jnp.float32),
        pltpu.VMEM_SHARED((_SP2,), jnp.float32),
        pltpu.SemaphoreType.DMA,
    ],
    compiler_params=pltpu.CompilerParams(needs_layout_passes=False),
)(_sc_body)


def kernel(entity_id, neighbor_ids, path_ids, edge_ids, W, Lw, Lb):
  n = neighbor_ids.astype(jnp.int32)
  p = path_ids.astype(jnp.int32)
  e = edge_ids.astype(jnp.int32)
  s = entity_id.astype(jnp.int32)
  rows_n = jnp.pad(n.reshape(10, 20), ((0, 0), (0, RPT - 20)))
  rows_p = jnp.stack([
      jnp.pad(p[0:17], (0, RPT - 17)),
      jnp.pad(p[17:34], (0, RPT - 17)),
      jnp.pad(p[34:50], (0, RPT - 16)),
  ])
  rows_e = jnp.pad(e.reshape(1, 20), ((0, 0), (0, RPT - 20)))
  rows_s = jnp.pad(s.reshape(1, 1), ((0, 0), (0, RPT - 1)))
  idx_mat = jnp.concatenate(
      [rows_n, rows_p, rows_e, rows_s, jnp.zeros((1, RPT), jnp.int32)],
      axis=0).reshape(16 * RPT)
  head = jnp.concatenate(
      [Lw.astype(jnp.float32).reshape(3), Lb.astype(jnp.float32).reshape(1),
       jnp.zeros((L - 4,), jnp.float32)])
  out, = _sc_kernel(idx_mat, W.astype(jnp.float32), head)
  return (out[0:1], out[L:L + 1])


# speedup vs baseline: 1.0676x; 1.0676x over previous
import functools

import jax
import jax.numpy as jnp
from jax import lax
from jax.experimental import pallas as pl
from jax.experimental.pallas import tpu as pltpu
from jax.experimental.pallas import tpu_sc as plsc

L = 16
NCH = 8
DIM = 128
RPT = 24
PART = (NCH + 1) * L
P3_TILE = 14
_FETCH = 10 * PART
_SP1 = 13 * PART + _FETCH
_PIE_OFF = 16 * L
_SP2 = _PIE_OFF + 3 * DIM


def _sc_body(idx_hbm, w_hbm, head_hbm, out_hbm,
             idx_v, rows_v, part_v, pie_v, slab_v, part2_v,
             slab2_v, head_v, out_v, spmem1, spmem2, sem):
  tid = lax.axis_index("s")
  cid = lax.axis_index("c")
  zero = jnp.zeros((L,), jnp.float32)
  isp3 = jnp.logical_and(tid == P3_TILE, cid == 0)

  cnt = jnp.where(tid < 10, 20,
                  jnp.where(tid < 12, 17,
                            jnp.where(tid == 12, 16,
                                      jnp.where(tid == 13, 20, 0))))
  lo = jnp.where(tid < 10, 0, jnp.where(tid < 13, 10, 13))
  grp = jnp.where(tid < 10, 10,
                  jnp.where(tid < 13, 3, jnp.where(tid == 13, 1, 0)))
  is_leader = jnp.logical_or(tid == 0, jnp.logical_or(tid == 10, tid == 13))
  leader_ix = jnp.where(tid == 0, 0, jnp.where(tid == 10, 1, 2))

  @pl.when(isp3)
  def _():
    pltpu.sync_copy(head_hbm, head_v)

  pltpu.sync_copy(idx_hbm.at[pl.ds(tid * RPT, RPT)], idx_v)
  pltpu.async_copy(w_hbm.at[idx_v], rows_v, sem).wait()

  def p1_body(r, carry):
    ch = [rows_v[r, pl.ds(c * L, L)] for c in range(NCH)]
    s = tuple(carry[c] + ch[c] for c in range(NCH))
    q = carry[NCH]
    for c in range(NCH):
      q = q + ch[c] * ch[c]
    return s + (q,)

  @pl.when(cnt > 0)
  def _():
    carry = lax.fori_loop(0, cnt, p1_body, (zero,) * (NCH + 1))
    for c in range(NCH + 1):
      part_v[pl.ds(c * L, L)] = carry[c]
    pltpu.sync_copy(part_v, spmem1.at[pl.ds(tid * PART, PART)])

  plsc.subcore_barrier()

  def rsqrt16(x):
    i = plsc.bitcast(x, jnp.int32)
    i = jnp.int32(0x5F3759DF) - lax.shift_right_logical(i, 1)
    y = plsc.bitcast(i, jnp.float32)
    for _ in range(3):
      y = y * (1.5 - 0.5 * x * y * y)
    return y

  def exp_dot(rv, r, p):
    d = rv[r, pl.ds(0, L)] * p[0]
    for c in range(1, NCH):
      d = d + rv[r, pl.ds(c * L, L)] * p[c]
    return jnp.exp(jnp.broadcast_to(jnp.sum(d), (L,)))

  @pl.when(cnt > 0)
  def _():
    pltpu.sync_copy(spmem1.at[pl.ds(lo * PART, _FETCH)], slab_v)

    def pie_body(s_, carry):
      return tuple(carry[c] + slab_v[pl.ds(s_ * PART + c * L, L)]
                   for c in range(NCH + 1))

    tot = lax.fori_loop(0, grp, pie_body, (zero,) * (NCH + 1))
    rinv = rsqrt16(jnp.broadcast_to(jnp.sum(tot[NCH]), (L,)))
    pie = tuple(tot[c] * rinv for c in range(NCH))

    acc = lax.fori_loop(0, cnt, lambda r, a: a + exp_dot(rows_v, r, pie), zero)
    part2_v[...] = acc
    cp = pltpu.async_copy(part2_v, spmem2.at[pl.ds(tid * L, L)], sem)

    @pl.when(is_leader)
    def _():
      for c in range(NCH):
        pie_v[pl.ds(c * L, L)] = pie[c]
      pltpu.sync_copy(pie_v, spmem2.at[pl.ds(_PIE_OFF + leader_ix * DIM, DIM)])

    cp.wait()

  plsc.subcore_barrier()

  @pl.when(isp3)
  def _():
    pltpu.sync_copy(spmem2, slab2_v)
    dn = slab2_v[pl.ds(0, L)]
    for t in range(1, 10):
      dn = dn + slab2_v[pl.ds(t * L, L)]
    dp = (slab2_v[pl.ds(10 * L, L)] + slab2_v[pl.ds(11 * L, L)]
          + slab2_v[pl.ds(12 * L, L)])
    de = slab2_v[pl.ds(13 * L, L)]

    def numer(li):
      pl_ = tuple(slab2_v[pl.ds(_PIE_OFF + li * DIM + c * L, L)]
                  for c in range(NCH))
      return exp_dot(rows_v, 0, pl_)

    pn = numer(0) / dn
    pp = numer(1) / dp
    pe = numer(2) / de
    lane = lax.iota(jnp.int32, L)
    one = zero + 1.0
    pvec = jnp.where(lane == 0, pn,
                     jnp.where(lane == 1, pp,
                               jnp.where(lane == 2, pe,
                                         jnp.where(lane == 3, one, zero))))
    gw = jnp.broadcast_to(jnp.sum(pvec * head_v[...]), (L,))
    out_v[pl.ds(0, L)] = gw
    out_v[pl.ds(L, L)] = 1.0 - gw
    pltpu.sync_copy(out_v, out_hbm)


_sc_kernel = functools.partial(
    pl.kernel,
    out_type=(jax.ShapeDtypeStruct((2 * L,), jnp.float32),),
    mesh=plsc.VectorSubcoreMesh(core_axis_name="c", subcore_axis_name="s",
                                num_cores=1, num_subcores=16),
    scratch_types=[
        pltpu.VMEM((RPT,), jnp.int32),
        pltpu.VMEM((RPT, DIM), jnp.float32),
        pltpu.VMEM((PART,), jnp.float32),
        pltpu.VMEM((DIM,), jnp.float32),
        pltpu.VMEM((_FETCH,), jnp.float32),
        pltpu.VMEM((L,), jnp.float32),
        pltpu.VMEM((_SP2,), jnp.float32),
        pltpu.VMEM((L,), jnp.float32),
        pltpu.VMEM((2 * L,), jnp.float32),
        pltpu.VMEM_SHARED((_SP1,), jnp.float32),
        pltpu.VMEM_SHARED((_SP2,), jnp.float32),
        pltpu.SemaphoreType.DMA,
    ],
    compiler_params=pltpu.CompilerParams(needs_layout_passes=False),
)(_sc_body)


def kernel(entity_id, neighbor_ids, path_ids, edge_ids, W, Lw, Lb):
  n = neighbor_ids.astype(jnp.int32)
  p = path_ids.astype(jnp.int32)
  e = edge_ids.astype(jnp.int32)
  s = entity_id.astype(jnp.int32)
  rows_n = jnp.pad(n.reshape(10, 20), ((0, 0), (0, RPT - 20)))
  rows_p = jnp.stack([
      jnp.pad(p[0:17], (0, RPT - 17)),
      jnp.pad(p[17:34], (0, RPT - 17)),
      jnp.pad(p[34:50], (0, RPT - 16)),
  ])
  rows_e = jnp.pad(e.reshape(1, 20), ((0, 0), (0, RPT - 20)))
  rows_s = jnp.pad(s.reshape(1, 1), ((0, 0), (0, RPT - 1)))
  idx_mat = jnp.concatenate(
      [rows_n, rows_p, rows_e, rows_s, jnp.zeros((1, RPT), jnp.int32)],
      axis=0).reshape(16 * RPT)
  head = jnp.concatenate(
      [Lw.astype(jnp.float32).reshape(3), Lb.astype(jnp.float32).reshape(1),
       jnp.zeros((L - 4,), jnp.float32)])
  out, = _sc_kernel(idx_mat, W.astype(jnp.float32), head)
  return (out[0:1], out[L:L + 1])
